# baseline (speedup 1.0000x reference)
import functools

import jax
import jax.numpy as jnp
from jax import lax
from jax.experimental import pallas as pl
from jax.experimental.pallas import tpu as pltpu
from jax.experimental.pallas import tpu_sc as plsc

_N = 10000
_E = 320000
_HID = 128
_CHUNK = 128
_NCHUNK = _E // _CHUNK
_NC = 2
_NS = 16
_NW = _NC * _NS
_KPT = 80
_EPAD = _NW * _KPT * _CHUNK
_RPT = 640
_ACCPAD = _RPT * _NS

_BN = 1000
_BE = 4000


def _ew_body(a_ref, w_ref, o_ref):
    o_ref[...] = jnp.dot(a_ref[...], w_ref[...],
                         preferred_element_type=jnp.float32)


def _edge_weights(ea128, wm):
    n = ea128.shape[0]
    return pl.pallas_call(
        _ew_body,
        grid=(n // _BE,),
        in_specs=[pl.BlockSpec((_BE, 128), lambda i: (i, 0)),
                  pl.BlockSpec((128, 8), lambda i: (0, 0))],
        out_specs=pl.BlockSpec((_BE, 8), lambda i: (i, 0)),
        out_shape=jax.ShapeDtypeStruct((n, 8), jnp.float32),
    )(ea128, wm)


def _mm_body(x_ref, w_ref, o_ref):
    o_ref[...] = jnp.dot(x_ref[...], w_ref[...],
                         preferred_element_type=jnp.float32)


def _matmul(x, w):
    n, k = x.shape
    m = w.shape[1]
    return pl.pallas_call(
        _mm_body,
        grid=(n // _BN,),
        in_specs=[pl.BlockSpec((_BN, k), lambda i: (i, 0)),
                  pl.BlockSpec((k, m), lambda i: (0, 0))],
        out_specs=pl.BlockSpec((_BN, m), lambda i: (i, 0)),
        out_shape=jax.ShapeDtypeStruct((n, m), jnp.float32),
    )(x, w)


def _pre_body(dp_ref, xw_ref, dis_ref, xws_ref):
    deg = dp_ref[:, 0:1] + dp_ref[:, 1:2] + 1.0
    dis = lax.rsqrt(deg)
    dis_ref[...] = dis
    xws_ref[...] = xw_ref[...] * dis


def _pre(dp, xw0):
    return pl.pallas_call(
        _pre_body,
        grid=(_N // _BN,),
        in_specs=[pl.BlockSpec((_BN, 2), lambda i: (i, 0)),
                  pl.BlockSpec((_BN, _HID), lambda i: (i, 0))],
        out_specs=[pl.BlockSpec((_BN, 1), lambda i: (i, 0)),
                   pl.BlockSpec((_BN, _HID), lambda i: (i, 0))],
        out_shape=[jax.ShapeDtypeStruct((_N, 1), jnp.float32),
                   jax.ShapeDtypeStruct((_N, _HID), jnp.float32)],
    )(dp, xw0)


def _mid_body(p0_ref, p1_ref, xw_ref, dis_ref, b_ref, w_ref,
              xw1_ref, xws1_ref):
    d = dis_ref[...]
    h = d * (p0_ref[...] + p1_ref[...]) + d * d * xw_ref[...] + b_ref[...]
    h = jnp.maximum(h, 0.0)
    xw1 = jnp.dot(h, w_ref[...], preferred_element_type=jnp.float32)
    xw1_ref[...] = xw1
    xws1_ref[...] = xw1 * d


def _mid(p0, p1, xw0, dis, b0, w1):
    return pl.pallas_call(
        _mid_body,
        grid=(_N // _BN,),
        in_specs=[pl.BlockSpec((_BN, _HID), lambda i: (i, 0)),
                  pl.BlockSpec((_BN, _HID), lambda i: (i, 0)),
                  pl.BlockSpec((_BN, _HID), lambda i: (i, 0)),
                  pl.BlockSpec((_BN, 1), lambda i: (i, 0)),
                  pl.BlockSpec((1, _HID), lambda i: (0, 0)),
                  pl.BlockSpec((_HID, _HID), lambda i: (0, 0))],
        out_specs=[pl.BlockSpec((_BN, _HID), lambda i: (i, 0)),
                   pl.BlockSpec((_BN, _HID), lambda i: (i, 0))],
        out_shape=[jax.ShapeDtypeStruct((_N, _HID), jnp.float32),
                   jax.ShapeDtypeStruct((_N, _HID), jnp.float32)],
    )(p0, p1, xw0, dis, b0, w1)


def _out_body(q0_ref, q1_ref, xw_ref, dis_ref, b_ref, wl_ref, bl_ref, o_ref):
    d = dis_ref[...]
    h = d * (q0_ref[...] + q1_ref[...]) + d * d * xw_ref[...] + b_ref[...]
    h = jnp.maximum(h, 0.0)
    o_ref[...] = jnp.dot(h, wl_ref[...],
                         preferred_element_type=jnp.float32) + bl_ref[...]


def _head(q0, q1, xw1, dis, b1, wl, bl):
    ncls = wl.shape[1]
    return pl.pallas_call(
        _out_body,
        grid=(_N // _BN,),
        in_specs=[pl.BlockSpec((_BN, _HID), lambda i: (i, 0)),
                  pl.BlockSpec((_BN, _HID), lambda i: (i, 0)),
                  pl.BlockSpec((_BN, _HID), lambda i: (i, 0)),
                  pl.BlockSpec((_BN, 1), lambda i: (i, 0)),
                  pl.BlockSpec((1, _HID), lambda i: (0, 0)),
                  pl.BlockSpec((_HID, ncls), lambda i: (0, 0)),
                  pl.BlockSpec((1, ncls), lambda i: (0, 0))],
        out_specs=pl.BlockSpec((_BN, ncls), lambda i: (i, 0)),
        out_shape=jax.ShapeDtypeStruct((_N, ncls), jnp.float32),
    )(q0, q1, xw1, dis, b1, wl, bl)


_MESH = dict(core_axis_name="c", subcore_axis_name="s")


def _sc_deg(dst3, ew3):
    @functools.partial(
        pl.kernel,
        out_type=[jax.ShapeDtypeStruct((_N,), jnp.float32),
                  jax.ShapeDtypeStruct((_N,), jnp.float32)],
        mesh=plsc.VectorSubcoreMesh(**_MESH),
        scratch_types=[
            pltpu.VMEM_SHARED((_ACCPAD,), jnp.float32),
            pltpu.VMEM((640,), jnp.float32),
            pltpu.VMEM((_KPT, _CHUNK), jnp.int32),
            pltpu.VMEM((_KPT, _CHUNK), jnp.float32),
            pltpu.VMEM((_N,), jnp.float32),
            pltpu.SemaphoreType.DMA,
        ],
    )
    def k(dst_hbm, ew_hbm, out0_hbm, out1_hbm, acc, zbuf, dbuf, ebuf, tbuf,
          sem):
        c = lax.axis_index("c")
        s = lax.axis_index("s")
        wid = c * _NS + s

        pltpu.sync_copy(dst_hbm.at[wid], dbuf)
        pltpu.sync_copy(ew_hbm.at[wid], ebuf)

        def zero(i, carry):
            zbuf[pl.ds(i * 16, 16)] = jnp.zeros((16,), jnp.float32)
            return carry
        lax.fori_loop(0, 40, zero, 0)
        pltpu.sync_copy(zbuf, acc.at[pl.ds(s * 640, 640)])
        plsc.subcore_barrier()

        def blk(t, carry):
            for j in range(8):
                kk = t * 8 + j
                pltpu.async_copy(ebuf.at[kk], acc.at[dbuf.at[kk]], sem,
                                 add=True)
            for j in range(8):
                kk = t * 8 + j
                pltpu.make_async_copy(ebuf.at[kk], acc.at[dbuf.at[kk]],
                                      sem).wait()
            return carry
        lax.fori_loop(0, _KPT // 8, blk, 0)
        plsc.subcore_barrier()

        @pl.when(s == 0)
        def _():
            pltpu.sync_copy(acc.at[pl.ds(0, _N)], tbuf)

            @pl.when(c == 0)
            def _():
                pltpu.sync_copy(tbuf, out0_hbm)

            @pl.when(c == 1)
            def _():
                pltpu.sync_copy(tbuf, out1_hbm)

    return k(dst3, ew3)


def _sc_msg(xws, src3, dst3, ew3):
    half = _KPT // 2

    @functools.partial(
        pl.kernel,
        out_type=[jax.ShapeDtypeStruct((_N, _HID), jnp.float32),
                  jax.ShapeDtypeStruct((_N, _HID), jnp.float32)],
        mesh=plsc.VectorSubcoreMesh(**_MESH),
        scratch_types=[
            pltpu.VMEM_SHARED((_ACCPAD, _HID), jnp.float32),
            pltpu.VMEM((half, _CHUNK), jnp.int32),
            pltpu.VMEM((half, _CHUNK), jnp.int32),
            pltpu.VMEM((half, _CHUNK), jnp.float32),
            pltpu.VMEM((_CHUNK, _HID), jnp.float32),
            pltpu.VMEM((_CHUNK, _HID), jnp.float32),
            pltpu.SemaphoreType.DMA,
            pltpu.SemaphoreType.DMA,
            pltpu.SemaphoreType.DMA,
            pltpu.SemaphoreType.DMA,
        ],
    )
    def k(xws_hbm, src_hbm, dst_hbm, ew_hbm, out0_hbm, out1_hbm,
          acc, sbuf, dbuf, ebuf, rows0, rows1, gsem0, gsem1, ssem0, ssem1):
        c = lax.axis_index("c")
        s = lax.axis_index("s")
        wid = c * _NS + s

        def scale(rows, cid):
            def grp(g, carry2):
                ev = ebuf[cid, pl.ds(g * 16, 16)]
                for l in range(16):
                    sv = ev[l]
                    for j in range(_HID // 16):
                        sl = pl.ds(j * 16, 16)
                        rows[g * 16 + l, sl] = rows[g * 16 + l, sl] * sv
                return carry2
            lax.fori_loop(0, _CHUNK // 16, grp, 0)

        def run_half(h):
            pltpu.sync_copy(src_hbm.at[wid, pl.ds(h * half, half)], sbuf)
            pltpu.sync_copy(dst_hbm.at[wid, pl.ds(h * half, half)], dbuf)
            pltpu.sync_copy(ew_hbm.at[wid, pl.ds(h * half, half)], ebuf)
            pltpu.async_copy(xws_hbm.at[sbuf.at[0]], rows0, gsem0)
            if h == 0:
                plsc.subcore_barrier()

            def body(k, rows, gsem, ssem, orows, ogsem, ossem):
                pltpu.make_async_copy(xws_hbm.at[sbuf.at[k]], rows,
                                      gsem).wait()
                scale(rows, k)
                pltpu.async_copy(rows, acc.at[dbuf.at[k]], ssem, add=True)

                @pl.when(k > 0)
                def _():
                    pltpu.make_async_copy(orows, acc.at[dbuf.at[k - 1]],
                                          ossem).wait()

                @pl.when(k < half - 1)
                def _():
                    pltpu.async_copy(xws_hbm.at[sbuf.at[k + 1]], orows, ogsem)

            def step(k, carry):
                @pl.when(k % 2 == 0)
                def _():
                    body(k, rows0, gsem0, ssem0, rows1, gsem1, ssem1)

                @pl.when(k % 2 == 1)
                def _():
                    body(k, rows1, gsem1, ssem1, rows0, gsem0, ssem0)
                return carry
            lax.fori_loop(0, half, step, 0)
            pltpu.make_async_copy(rows1, acc.at[dbuf.at[half - 1]],
                                  ssem1).wait()

        def zero(i, carry):
            for j in range(_HID // 16):
                rows0[i, pl.ds(j * 16, 16)] = jnp.zeros((16,), jnp.float32)
            return carry
        lax.fori_loop(0, _CHUNK, zero, 0)
        for r in range(_RPT // _CHUNK):
            pltpu.sync_copy(rows0, acc.at[pl.ds(s * _RPT + r * _CHUNK,
                                                _CHUNK)])
        run_half(0)
        run_half(1)
        plsc.subcore_barrier()

        wbuf = rows0.at[pl.ds(0, 80)]
        for r in range(_RPT // 80):
            base = s * _RPT + r * 80

            @pl.when(base < _N)
            def _():
                bofs = pl.multiple_of(base, 8)
                pltpu.sync_copy(acc.at[pl.ds(bofs, 80)], wbuf)

                @pl.when(c == 0)
                def _():
                    pltpu.sync_copy(wbuf, out0_hbm.at[pl.ds(bofs, 80)])

                @pl.when(c == 1)
                def _():
                    pltpu.sync_copy(wbuf, out1_hbm.at[pl.ds(bofs, 80)])

    return k(xws, src3, dst3, ew3)


def kernel(x, edge_index, edge_attr, W_ew, W0, b0, W1, b1, Wl, bl):
    src1d = edge_index[0].astype(jnp.int32)
    dst1d = edge_index[1].astype(jnp.int32)

    wm = jnp.kron(jnp.eye(8, dtype=jnp.float32), W_ew)
    ew = _edge_weights(edge_attr.reshape(_E // 8, 128), wm).reshape(_E)

    pad = _EPAD - _E
    fill = (jnp.arange(pad, dtype=jnp.int32) * 97) % _N
    src3 = jnp.concatenate([src1d, fill]).reshape(_NW, _KPT, _CHUNK)
    dst3 = jnp.concatenate([dst1d, fill]).reshape(_NW, _KPT, _CHUNK)
    ew3 = jnp.concatenate([ew, jnp.zeros((pad,), jnp.float32)]
                          ).reshape(_NW, _KPT, _CHUNK)

    dp0, dp1 = _sc_deg(dst3, ew3)
    xw0 = _matmul(x, W0)
    dis, xws0 = _pre(jnp.stack([dp0, dp1], axis=1), xw0)

    p0, p1 = _sc_msg(xws0, src3, dst3, ew3)
    xw1, xws1 = _mid(p0, p1, xw0, dis, b0.reshape(1, -1), W1)

    q0, q1 = _sc_msg(xws1, src3, dst3, ew3)
    out = _head(q0, q1, xw1, dis, b1.reshape(1, -1), Wl, bl.reshape(1, -1))
    return out


# speedup vs baseline: 1.1514x; 1.1514x over previous
import functools

import jax
import jax.numpy as jnp
from jax import lax
from jax.experimental import pallas as pl
from jax.experimental.pallas import tpu as pltpu
from jax.experimental.pallas import tpu_sc as plsc

_N = 10000
_E = 320000
_HID = 128
_CHUNK = 128
_NCHUNK = _E // _CHUNK
_NC = 2
_NS = 16
_NW = _NC * _NS
_KPT = 80
_EPAD = _NW * _KPT * _CHUNK
_RPT = 640
_ACCPAD = _RPT * _NS

_BN = 1000
_BE = 4096


def _ew_body(a_ref, w_ref, o_ref):
    o_ref[...] = jnp.dot(a_ref[...], w_ref[...],
                         preferred_element_type=jnp.float32).T


def _edge_weights(ea128, wm):
    n = ea128.shape[0]
    return pl.pallas_call(
        _ew_body,
        grid=(pl.cdiv(n, _BE),),
        in_specs=[pl.BlockSpec((_BE, 128), lambda i: (i, 0)),
                  pl.BlockSpec((128, 8), lambda i: (0, 0))],
        out_specs=pl.BlockSpec((8, _BE), lambda i: (0, i)),
        out_shape=jax.ShapeDtypeStruct((8, n), jnp.float32),
    )(ea128, wm)


def _mm_body(x_ref, w_ref, o_ref):
    o_ref[...] = jnp.dot(x_ref[...], w_ref[...],
                         preferred_element_type=jnp.float32)


def _matmul(x, w):
    n, k = x.shape
    m = w.shape[1]
    return pl.pallas_call(
        _mm_body,
        grid=(n // _BN,),
        in_specs=[pl.BlockSpec((_BN, k), lambda i: (i, 0)),
                  pl.BlockSpec((k, m), lambda i: (0, 0))],
        out_specs=pl.BlockSpec((_BN, m), lambda i: (i, 0)),
        out_shape=jax.ShapeDtypeStruct((n, m), jnp.float32),
    )(x, w)


def _pre_body(dp_ref, xw_ref, dis_ref, xws_ref):
    deg = dp_ref[:, 0:1] + dp_ref[:, 1:2] + 1.0
    dis = lax.rsqrt(deg)
    dis_ref[...] = dis
    xws_ref[...] = xw_ref[...] * dis


def _pre(dp, xw0):
    return pl.pallas_call(
        _pre_body,
        grid=(_N // _BN,),
        in_specs=[pl.BlockSpec((_BN, 2), lambda i: (i, 0)),
                  pl.BlockSpec((_BN, _HID), lambda i: (i, 0))],
        out_specs=[pl.BlockSpec((_BN, 1), lambda i: (i, 0)),
                   pl.BlockSpec((_BN, _HID), lambda i: (i, 0))],
        out_shape=[jax.ShapeDtypeStruct((_N, 1), jnp.float32),
                   jax.ShapeDtypeStruct((_N, _HID), jnp.float32)],
    )(dp, xw0)


def _mid_body(p0_ref, p1_ref, xw_ref, dis_ref, b_ref, w_ref,
              xw1_ref, xws1_ref):
    d = dis_ref[...]
    h = d * (p0_ref[...] + p1_ref[...]) + d * d * xw_ref[...] + b_ref[...]
    h = jnp.maximum(h, 0.0)
    xw1 = jnp.dot(h, w_ref[...], preferred_element_type=jnp.float32)
    xw1_ref[...] = xw1
    xws1_ref[...] = xw1 * d


def _mid(p0, p1, xw0, dis, b0, w1):
    return pl.pallas_call(
        _mid_body,
        grid=(_N // _BN,),
        in_specs=[pl.BlockSpec((_BN, _HID), lambda i: (i, 0)),
                  pl.BlockSpec((_BN, _HID), lambda i: (i, 0)),
                  pl.BlockSpec((_BN, _HID), lambda i: (i, 0)),
                  pl.BlockSpec((_BN, 1), lambda i: (i, 0)),
                  pl.BlockSpec((1, _HID), lambda i: (0, 0)),
                  pl.BlockSpec((_HID, _HID), lambda i: (0, 0))],
        out_specs=[pl.BlockSpec((_BN, _HID), lambda i: (i, 0)),
                   pl.BlockSpec((_BN, _HID), lambda i: (i, 0))],
        out_shape=[jax.ShapeDtypeStruct((_N, _HID), jnp.float32),
                   jax.ShapeDtypeStruct((_N, _HID), jnp.float32)],
    )(p0, p1, xw0, dis, b0, w1)


def _out_body(q0_ref, q1_ref, xw_ref, dis_ref, b_ref, wl_ref, bl_ref, o_ref):
    d = dis_ref[...]
    h = d * (q0_ref[...] + q1_ref[...]) + d * d * xw_ref[...] + b_ref[...]
    h = jnp.maximum(h, 0.0)
    o_ref[...] = jnp.dot(h, wl_ref[...],
                         preferred_element_type=jnp.float32) + bl_ref[...]


def _head(q0, q1, xw1, dis, b1, wl, bl):
    ncls = wl.shape[1]
    return pl.pallas_call(
        _out_body,
        grid=(_N // _BN,),
        in_specs=[pl.BlockSpec((_BN, _HID), lambda i: (i, 0)),
                  pl.BlockSpec((_BN, _HID), lambda i: (i, 0)),
                  pl.BlockSpec((_BN, _HID), lambda i: (i, 0)),
                  pl.BlockSpec((_BN, 1), lambda i: (i, 0)),
                  pl.BlockSpec((1, _HID), lambda i: (0, 0)),
                  pl.BlockSpec((_HID, ncls), lambda i: (0, 0)),
                  pl.BlockSpec((1, ncls), lambda i: (0, 0))],
        out_specs=pl.BlockSpec((_BN, ncls), lambda i: (i, 0)),
        out_shape=jax.ShapeDtypeStruct((_N, ncls), jnp.float32),
    )(q0, q1, xw1, dis, b1, wl, bl)


_MESH = dict(core_axis_name="c", subcore_axis_name="s")


def _sc_deg(dst3, ew3):
    @functools.partial(
        pl.kernel,
        out_type=[jax.ShapeDtypeStruct((_N,), jnp.float32),
                  jax.ShapeDtypeStruct((_N,), jnp.float32)],
        mesh=plsc.VectorSubcoreMesh(**_MESH),
        scratch_types=[
            pltpu.VMEM_SHARED((_ACCPAD,), jnp.float32),
            pltpu.VMEM((640,), jnp.float32),
            pltpu.VMEM((_KPT, _CHUNK), jnp.int32),
            pltpu.VMEM((_KPT, _CHUNK), jnp.float32),
            pltpu.VMEM((_N,), jnp.float32),
            pltpu.SemaphoreType.DMA,
        ],
    )
    def k(dst_hbm, ew_hbm, out0_hbm, out1_hbm, acc, zbuf, dbuf, ebuf, tbuf,
          sem):
        c = lax.axis_index("c")
        s = lax.axis_index("s")
        wid = c * _NS + s

        pltpu.sync_copy(dst_hbm.at[wid], dbuf)
        pltpu.sync_copy(ew_hbm.at[wid], ebuf)

        def zero(i, carry):
            zbuf[pl.ds(i * 16, 16)] = jnp.zeros((16,), jnp.float32)
            return carry
        lax.fori_loop(0, 40, zero, 0)
        pltpu.sync_copy(zbuf, acc.at[pl.ds(s * 640, 640)])
        plsc.subcore_barrier()

        def blk(t, carry):
            for j in range(8):
                kk = t * 8 + j
                pltpu.async_copy(ebuf.at[kk], acc.at[dbuf.at[kk]], sem,
                                 add=True)
            for j in range(8):
                kk = t * 8 + j
                pltpu.make_async_copy(ebuf.at[kk], acc.at[dbuf.at[kk]],
                                      sem).wait()
            return carry
        lax.fori_loop(0, _KPT // 8, blk, 0)
        plsc.subcore_barrier()

        @pl.when(s == 0)
        def _():
            pltpu.sync_copy(acc.at[pl.ds(0, _N)], tbuf)

            @pl.when(c == 0)
            def _():
                pltpu.sync_copy(tbuf, out0_hbm)

            @pl.when(c == 1)
            def _():
                pltpu.sync_copy(tbuf, out1_hbm)

    return k(dst3, ew3)


def _sc_msg(xws, src3, dst3, ew3):
    half = _KPT // 2

    @functools.partial(
        pl.kernel,
        out_type=[jax.ShapeDtypeStruct((_N, _HID), jnp.float32),
                  jax.ShapeDtypeStruct((_N, _HID), jnp.float32)],
        mesh=plsc.VectorSubcoreMesh(**_MESH),
        scratch_types=[
            pltpu.VMEM_SHARED((_ACCPAD, _HID), jnp.float32),
            pltpu.VMEM((half, _CHUNK), jnp.int32),
            pltpu.VMEM((half, _CHUNK), jnp.int32),
            pltpu.VMEM((half, _CHUNK), jnp.float32),
            pltpu.VMEM((_CHUNK, _HID), jnp.float32),
            pltpu.VMEM((_CHUNK, _HID), jnp.float32),
            pltpu.SemaphoreType.DMA,
            pltpu.SemaphoreType.DMA,
            pltpu.SemaphoreType.DMA,
            pltpu.SemaphoreType.DMA,
        ],
    )
    def k(xws_hbm, src_hbm, dst_hbm, ew_hbm, out0_hbm, out1_hbm,
          acc, sbuf, dbuf, ebuf, rows0, rows1, gsem0, gsem1, ssem0, ssem1):
        c = lax.axis_index("c")
        s = lax.axis_index("s")
        wid = c * _NS + s

        def scale(rows, cid):
            def grp(g, carry2):
                ev = ebuf[cid, pl.ds(g * 16, 16)]
                for l in range(16):
                    sv = ev[l]
                    for j in range(_HID // 16):
                        sl = pl.ds(j * 16, 16)
                        rows[g * 16 + l, sl] = rows[g * 16 + l, sl] * sv
                return carry2
            lax.fori_loop(0, _CHUNK // 16, grp, 0)

        def run_half(h):
            pltpu.sync_copy(src_hbm.at[wid, pl.ds(h * half, half)], sbuf)
            pltpu.sync_copy(dst_hbm.at[wid, pl.ds(h * half, half)], dbuf)
            pltpu.sync_copy(ew_hbm.at[wid, pl.ds(h * half, half)], ebuf)
            pltpu.async_copy(xws_hbm.at[sbuf.at[0]], rows0, gsem0)
            pltpu.async_copy(xws_hbm.at[sbuf.at[1]], rows1, gsem1)
            if h == 0:
                plsc.subcore_barrier()

            def pair(t, carry):
                a = 2 * t
                b = 2 * t + 1
                pltpu.make_async_copy(xws_hbm.at[sbuf.at[a]], rows0,
                                      gsem0).wait()
                scale(rows0, a)
                pltpu.async_copy(rows0, acc.at[dbuf.at[a]], ssem0, add=True)
                pltpu.make_async_copy(xws_hbm.at[sbuf.at[b]], rows1,
                                      gsem1).wait()
                pltpu.make_async_copy(rows0, acc.at[dbuf.at[a]], ssem0).wait()

                @pl.when(t < half // 2 - 1)
                def _():
                    pltpu.async_copy(xws_hbm.at[sbuf.at[a + 2]], rows0, gsem0)
                scale(rows1, b)
                pltpu.async_copy(rows1, acc.at[dbuf.at[b]], ssem1, add=True)
                pltpu.make_async_copy(rows1, acc.at[dbuf.at[b]], ssem1).wait()

                @pl.when(t < half // 2 - 1)
                def _():
                    pltpu.async_copy(xws_hbm.at[sbuf.at[b + 2]], rows1, gsem1)
                return carry
            lax.fori_loop(0, half // 2, pair, 0)

        def zero(i, carry):
            for j in range(_HID // 16):
                rows0[i, pl.ds(j * 16, 16)] = jnp.zeros((16,), jnp.float32)
            return carry
        lax.fori_loop(0, _CHUNK, zero, 0)
        for r in range(_RPT // _CHUNK):
            pltpu.sync_copy(rows0, acc.at[pl.ds(s * _RPT + r * _CHUNK,
                                                _CHUNK)])
        run_half(0)
        run_half(1)
        plsc.subcore_barrier()

        wbuf = rows0.at[pl.ds(0, 80)]
        for r in range(_RPT // 80):
            base = s * _RPT + r * 80

            @pl.when(base < _N)
            def _():
                bofs = pl.multiple_of(base, 8)
                pltpu.sync_copy(acc.at[pl.ds(bofs, 80)], wbuf)

                @pl.when(c == 0)
                def _():
                    pltpu.sync_copy(wbuf, out0_hbm.at[pl.ds(bofs, 80)])

                @pl.when(c == 1)
                def _():
                    pltpu.sync_copy(wbuf, out1_hbm.at[pl.ds(bofs, 80)])

    return k(xws, src3, dst3, ew3)


def kernel(x, edge_index, edge_attr, W_ew, W0, b0, W1, b1, Wl, bl):
    src1d = edge_index[0].astype(jnp.int32)
    dst1d = edge_index[1].astype(jnp.int32)

    wm = jnp.kron(jnp.eye(8, dtype=jnp.float32), W_ew)
    ew = _edge_weights(edge_attr.reshape(_E // 8, 128), wm).T.reshape(_E)

    pad = _EPAD - _E
    fill = (jnp.arange(pad, dtype=jnp.int32) * 97) % _N
    src3 = jnp.concatenate([src1d, fill]).reshape(_NW, _KPT, _CHUNK)
    dst3 = jnp.concatenate([dst1d, fill]).reshape(_NW, _KPT, _CHUNK)
    ew3 = jnp.concatenate([ew, jnp.zeros((pad,), jnp.float32)]
                          ).reshape(_NW, _KPT, _CHUNK)

    dp0, dp1 = _sc_deg(dst3, ew3)
    xw0 = _matmul(x, W0)
    dis, xws0 = _pre(jnp.stack([dp0, dp1], axis=1), xw0)

    p0, p1 = _sc_msg(xws0, src3, dst3, ew3)
    xw1, xws1 = _mid(p0, p1, xw0, dis, b0.reshape(1, -1), W1)

    q0, q1 = _sc_msg(xws1, src3, dst3, ew3)
    out = _head(q0, q1, xw1, dis, b1.reshape(1, -1), Wl, bl.reshape(1, -1))
    return out
